# baseline (speedup 1.0000x reference)
import functools

import jax
import jax.numpy as jnp
from jax import lax
from jax.experimental import pallas as pl
from jax.experimental.pallas import tpu as pltpu
from jax.experimental.pallas import tpu_sc as plsc

_B, _P, _O, _C = 16, 8732, 12, 21
_THR, _RATIO, _THETA, _ALPHA = 0.5, 3, 0.01, 1.0
_SL, _LN = 8, 1152
_PP = _SL * _LN


def _diou(px0, py0, px1, py1, tx0, ty0, tx1, ty1):
    eps = 1e-7
    ix0 = jnp.maximum(px0, tx0)
    iy0 = jnp.maximum(py0, ty0)
    ix1 = jnp.minimum(px1, tx1)
    iy1 = jnp.minimum(py1, ty1)
    inter = jnp.clip(ix1 - ix0, 0.0, None) * jnp.clip(iy1 - iy0, 0.0, None)
    ap = jnp.clip(px1 - px0, 0.0, None) * jnp.clip(py1 - py0, 0.0, None)
    at = jnp.clip(tx1 - tx0, 0.0, None) * jnp.clip(ty1 - ty0, 0.0, None)
    union = ap + at - inter
    iou = inter / (union + eps)
    cpx = (px0 + px1) / 2
    cpy = (py0 + py1) / 2
    ctx = (tx0 + tx1) / 2
    cty = (ty0 + ty1) / 2
    rho2 = (cpx - ctx) ** 2 + (cpy - cty) ** 2
    ex0 = jnp.minimum(px0, tx0)
    ey0 = jnp.minimum(py0, ty0)
    ex1 = jnp.maximum(px1, tx1)
    ey1 = jnp.maximum(py1, ty1)
    c2 = (ex1 - ex0) ** 2 + (ey1 - ey0) ** 2 + eps
    return 1.0 - (iou - rho2 / c2)


def _iotas():
    shp = (_SL, _LN)
    row = lax.broadcasted_iota(jnp.int32, shp, 0)
    coli = lax.broadcasted_iota(jnp.int32, shp, 1)
    pidx = row * _LN + coli
    return pidx, pidx >= _P


def _priors_xy(pr_ref):
    pcx, pcy, pw, ph = pr_ref[0], pr_ref[1], pr_ref[2], pr_ref[3]
    px0 = pcx - pw / 2
    py0 = pcy - ph / 2
    px1 = pcx + pw / 2
    py1 = pcy + ph / 2
    return pcx, pcy, pw, ph, px0, py0, px1, py1


def _arm_decode(al_ref, pr):
    pcx, pcy, pw, ph = pr[0], pr[1], pr[2], pr[3]
    g0, g1, g2, g3 = al_ref[0, 0], al_ref[0, 1], al_ref[0, 2], al_ref[0, 3]
    acx = g0 * pw / 10 + pcx
    acy = g1 * ph / 10 + pcy
    aw = jnp.exp(g2 / 5) * pw
    ah = jnp.exp(g3 / 5) * ph
    return acx - aw / 2, acy - ah / 2, acx + aw / 2, acy + ah / 2


def _read_objs(bx_ref, lb_ref):
    boxes = [[bx_ref[0, o, j] for j in range(4)] for o in range(_O)]
    labels = [lb_ref[0, 0, o] for o in range(_O)]
    return boxes, labels


_NV = _LN // 128


def _bcast_reduce(x, op):
    r = x[:, 0:128]
    for k in range(1, _NV):
        r = op(r, x[:, 128 * k:128 * (k + 1)])
    for s in (1, 2, 4, 8, 16, 32, 64):
        r = op(r, pltpu.roll(r, s, axis=1))
    for s in (1, 2, 4):
        r = op(r, pltpu.roll(r, s, axis=0))
    return jnp.concatenate([r] * _NV, axis=1)


def _run_match(boxes, labels, pidx, x0, y0, x1, y1, thr):
    shp = (_SL, _LN)
    area2 = (x1 - x0) * (y1 - y0)
    best = jnp.full(shp, -1.0, jnp.float32)
    obj = jnp.zeros(shp, jnp.int32)
    mx_l, pf_l = [], []
    for o in range(_O):
        bx0, by0, bx1, by1 = boxes[o]
        a1 = (bx1 - bx0) * (by1 - by0)
        inter = (jnp.clip(jnp.minimum(bx1, x1) - jnp.maximum(bx0, x0), 0.0, None)
                 * jnp.clip(jnp.minimum(by1, y1) - jnp.maximum(by0, y0), 0.0, None))
        ov = inter / jnp.maximum(a1 + area2 - inter, 1e-10)
        upd = ov > best
        best = jnp.where(upd, ov, best)
        obj = jnp.where(upd, o, obj)
        mx = _bcast_reduce(ov, jnp.maximum)
        pf = _bcast_reduce(jnp.where(ov == mx, pidx, _PP), jnp.minimum)
        mx_l.append(mx)
        pf_l.append(pf)
    obj0 = obj
    ofp = best
    for o in range(_O):
        m = mx_l[o] > 0.0
        hit = (pidx == pf_l[o]) & m
        ofp = jnp.where(hit, jnp.maximum(ofp, 1.0), ofp)
        obj = jnp.where(hit, o, jnp.where((pidx == pf_l[o]) & (~m), obj0, obj))
    lfp = jnp.zeros(shp, jnp.int32)
    tx0 = jnp.zeros(shp, jnp.float32)
    ty0 = jnp.zeros(shp, jnp.float32)
    tx1 = jnp.zeros(shp, jnp.float32)
    ty1 = jnp.zeros(shp, jnp.float32)
    for o in range(_O):
        sel = obj == o
        lfp = jnp.where(sel, labels[o], lfp)
        tx0 = jnp.where(sel, boxes[o][0], tx0)
        ty0 = jnp.where(sel, boxes[o][1], ty0)
        tx1 = jnp.where(sel, boxes[o][2], tx1)
        ty1 = jnp.where(sel, boxes[o][3], ty1)
    lfp = jnp.where(ofp < thr, 0, lfp)
    return lfp, (tx0, ty0, tx1, ty1)


def _write_part(part_ref, vals):
    li = lax.broadcasted_iota(jnp.int32, (1, 128), 1)
    acc = jnp.zeros((1, 128), jnp.float32)
    for j, v in enumerate(vals):
        acc = jnp.where(li == j, v, acc)
    part_ref[0] = acc


def _arm_body(al_ref, as_ref, pr_ref, bx_ref, lb_ref,
              conf_ref, bits_ref, part_ref):
    pidx, pad = _iotas()
    pr = _priors_xy(pr_ref)
    px0, py0, px1, py1 = pr[4], pr[5], pr[6], pr[7]
    ax0, ay0, ax1, ay1 = _arm_decode(al_ref, pr)
    boxes, labels = _read_objs(bx_ref, lb_ref)

    lfp_a, tla = _run_match(boxes, labels, pidx, px0, py0, px1, py1,
                            _THR - 0.2)
    pos_a = lfp_a > 0
    n_pos_a = jnp.sum(jnp.where(pos_a, 1.0, 0.0))
    d_a = _diou(ax0, ay0, ax1, ay1, *tla)
    dsum_a = jnp.sum(jnp.where(pos_a, d_a, 0.0))
    s0, s1 = as_ref[0, 0], as_ref[0, 1]
    mx2 = jnp.maximum(s0, s1)
    lse2 = mx2 + jnp.log(jnp.exp(s0 - mx2) + jnp.exp(s1 - mx2))
    ce_a = lse2 - jnp.where(pos_a, s1, s0)
    pos_ce_a = jnp.sum(jnp.where(pos_a, ce_a, 0.0))
    conf_na = jnp.where(pos_a | pad, 0.0, ce_a)
    conf_ref[0] = conf_na
    bits_ref[0] = lax.bitcast_convert_type(conf_na, jnp.int32)
    _write_part(part_ref, [n_pos_a, pos_ce_a, dsum_a])


def _odm_body(al_ref, as_ref, ol_ref, os_ref, pr_ref, bx_ref, lb_ref,
              conf_ref, bits_ref, part_ref):
    pidx, pad = _iotas()
    pr = _priors_xy(pr_ref)
    ax0, ay0, ax1, ay1 = _arm_decode(al_ref, pr)
    boxes, labels = _read_objs(bx_ref, lb_ref)

    lfp_o, tlo = _run_match(boxes, labels, pidx, ax0, ay0, ax1, ay1, _THR)
    s0, s1 = as_ref[0, 0], as_ref[0, 1]
    mx2 = jnp.maximum(s0, s1)
    lse2 = mx2 + jnp.log(jnp.exp(s0 - mx2) + jnp.exp(s1 - mx2))
    easy = jnp.exp(s1 - lse2) < _THETA
    pos_o = (lfp_o > 0) & (~easy)
    n_pos_o = jnp.sum(jnp.where(pos_o, 1.0, 0.0))

    acx2 = (ax0 + ax1) / 2
    acy2 = (ay0 + ay1) / 2
    aw2 = ax1 - ax0
    ah2 = ay1 - ay0
    h0, h1, h2, h3 = ol_ref[0, 0], ol_ref[0, 1], ol_ref[0, 2], ol_ref[0, 3]
    ocx = h0 * aw2 / 10 + acx2
    ocy = h1 * ah2 / 10 + acy2
    ow = jnp.exp(h2 / 5) * aw2
    oh = jnp.exp(h3 / 5) * ah2
    d_o = _diou(ocx - ow / 2, ocy - oh / 2, ocx + ow / 2, ocy + oh / 2, *tlo)
    dsum_o = jnp.sum(jnp.where(pos_o, d_o, 0.0))

    logits = [os_ref[0, c] for c in range(_C)]
    mxc = logits[0]
    for c in range(1, _C):
        mxc = jnp.maximum(mxc, logits[c])
    sume = jnp.exp(logits[0] - mxc)
    for c in range(1, _C):
        sume = sume + jnp.exp(logits[c] - mxc)
    lsec = mxc + jnp.log(sume)
    chosen = jnp.zeros((_SL, _LN), jnp.float32)
    for c in range(_C):
        chosen = jnp.where(lfp_o == c, logits[c], chosen)
    ce_o = lsec - chosen
    pos_ce_o = jnp.sum(jnp.where(pos_o, ce_o, 0.0))
    conf_no = jnp.where(pos_o | easy | pad, 0.0, ce_o)
    conf_ref[0] = conf_no
    bits_ref[0] = lax.bitcast_convert_type(conf_no, jnp.int32)
    _write_part(part_ref, [n_pos_o, pos_ce_o, dsum_o])


_UN = 8


def _sc_mine_body(cfa_hbm, cia_hbm, pa_hbm, cfo_hbm, cio_hbm, po_hbm, out_hbm,
                  row_f, row_i, part_v, out_v):
    wid = lax.axis_index("s") * 2 + lax.axis_index("c")
    is_arm = wid < _B
    img = jnp.where(is_arm, wid, wid - _B)

    @pl.when(is_arm)
    def _():
        pltpu.sync_copy(cfa_hbm.at[pl.ds(img * _PP, _PP)], row_f)
        pltpu.sync_copy(cia_hbm.at[pl.ds(img * _PP, _PP)], row_i)
        pltpu.sync_copy(pa_hbm.at[pl.ds(img * 128, 16)], part_v)

    @pl.when(jnp.logical_not(is_arm))
    def _():
        pltpu.sync_copy(cfo_hbm.at[pl.ds(img * _PP, _PP)], row_f)
        pltpu.sync_copy(cio_hbm.at[pl.ds(img * _PP, _PP)], row_i)
        pltpu.sync_copy(po_hbm.at[pl.ds(img * 128, 16)], part_v)

    if True:
        lane = lax.iota(jnp.int32, 16)

        def _shuf(x, s):
            return x.at[lane ^ s].get(mode="promise_in_bounds")

        def allsum(x):
            for s in (8, 4, 2, 1):
                x = x + _shuf(x, s)
            return x

        def allmax(x):
            for s in (8, 4, 2, 1):
                x = jnp.maximum(x, _shuf(x, s))
            return x

        def tree(xs, op):
            xs = list(xs)
            while len(xs) > 1:
                xs = [op(xs[i], xs[i + 1]) for i in range(0, len(xs) - 1, 2)] + \
                    (xs[-1:] if len(xs) % 2 else [])
            return xs[0]

        kvec = 3.0 * allsum(jnp.where(lane == 0, part_v[...], 0.0))

        zi = jnp.zeros((16,), jnp.int32)
        zf = jnp.zeros((16,), jnp.float32)

        @plsc.parallel_loop(0, _PP, 16 * _UN, carry=(zi,) * _UN)
        def mxs(j, c):
            return tuple(jnp.maximum(c[i], row_i[pl.ds(j + 16 * i, 16)])
                         for i in range(_UN))

        hi = allmax(tree(mxs, jnp.maximum)) + 1
        lo = zi

        def bisect(_, c):
            lo, hi = c
            mid = lo + ((hi - lo) >> 1)

            @plsc.parallel_loop(0, _PP, 16 * _UN, carry=(zf,) * _UN)
            def accs(j, c):
                return tuple(
                    c[i] + jnp.where(row_i[pl.ds(j + 16 * i, 16)] > mid,
                                     1.0, 0.0)
                    for i in range(_UN))

            p = allsum(tree(accs, jnp.add)) < kvec
            return jnp.where(p, lo, mid + 1), jnp.where(p, mid, hi)

        lo, hi = lax.fori_loop(0, 31, bisect, (lo, hi))

        @plsc.parallel_loop(0, _PP, 16 * 4,
                            carry=((zf,) * 4, (zf,) * 4, (zf,) * 4))
        def fin(j, c):
            acc, sab, tac = c
            na, ns, nt = [], [], []
            for i in range(4):
                b = row_i[pl.ds(j + 16 * i, 16)]
                v = row_f[pl.ds(j + 16 * i, 16)]
                gt = b > lo
                na.append(acc[i] + jnp.where(gt, 1.0, 0.0))
                ns.append(sab[i] + jnp.where(gt, v, 0.0))
                nt.append(jnp.maximum(tac[i], jnp.where(b == lo, v, 0.0)))
            return tuple(na), tuple(ns), tuple(nt)

        acc, sab, tac = fin
        t = allmax(tree(tac, jnp.maximum))
        neg = allsum(tree(sab, jnp.add)) + \
            (kvec - allsum(tree(acc, jnp.add))) * t
        out_v[...] = jnp.where(lane == 0, neg, 0.0)
        pltpu.sync_copy(out_v, out_hbm.at[pl.ds(wid * 16, 16)])


def _combine_body(pa_ref, po_ref, na_ref, no_ref, out_ref):
    li = lax.broadcasted_iota(jnp.int32, (_B, 128), 1)

    def col(ref, j):
        return jnp.sum(jnp.where(li == j, ref[:, 0, :], 0.0))

    npa, cepa, da = col(pa_ref, 0), col(pa_ref, 1), col(pa_ref, 2)
    npo, cepo, do_ = col(po_ref, 0), col(po_ref, 1), col(po_ref, 2)
    ci = lax.broadcasted_iota(jnp.int32, (_B, 16), 1)
    neg_a = jnp.sum(jnp.where(ci == 0, na_ref[...], 0.0))
    neg_o = jnp.sum(jnp.where(ci == 0, no_ref[...], 0.0))
    conf_a = (neg_a + cepa) / npa
    loc_a = da / jnp.maximum(npa, 1.0)
    conf_o = (neg_o + cepo) / npo
    loc_o = do_ / jnp.maximum(npo, 1.0)
    out_ref[0, 0] = conf_a + _ALPHA * loc_a + conf_o + _ALPHA * loc_o


def _prep(x):
    b, p, k = x.shape
    xt = jnp.swapaxes(x, 1, 2)
    xt = jnp.pad(xt, ((0, 0), (0, 0), (0, _PP - p)))
    return xt.reshape(b, k, _SL, _LN)


def _sc_mine(conf_a, bits_a, part_a, conf_o, bits_o, part_o):
    mesh = plsc.VectorSubcoreMesh(core_axis_name="c", subcore_axis_name="s",
                                  num_cores=2, num_subcores=16)
    return pl.kernel(
        _sc_mine_body,
        out_type=jax.ShapeDtypeStruct((2 * _B * 16,), jnp.float32),
        mesh=mesh,
        scratch_types=[
            pltpu.VMEM((_PP,), jnp.float32),
            pltpu.VMEM((_PP,), jnp.int32),
            pltpu.VMEM((16,), jnp.float32),
            pltpu.VMEM((16,), jnp.float32),
        ],
    )(conf_a.reshape(-1), bits_a.reshape(-1), part_a.reshape(-1),
      conf_o.reshape(-1), bits_o.reshape(-1), part_o.reshape(-1))


@jax.jit
def kernel(arm_locs, arm_scores, odm_locs, odm_scores, boxes, labels, priors_cxcy):
    al = _prep(arm_locs)
    asr = _prep(arm_scores)
    ol = _prep(odm_locs)
    osr = _prep(odm_scores)
    pr = jnp.pad(jnp.swapaxes(priors_cxcy, 0, 1),
                 ((0, 0), (0, _PP - _P))).reshape(4, _SL, _LN)
    labels = labels.astype(jnp.int32).reshape(_B, 1, _O)

    spec_pp = pl.BlockSpec((1, _SL, _LN), lambda b: (b, 0, 0))
    spec_part = pl.BlockSpec((1, 1, 128), lambda b: (b, 0, 0))
    spec_pr = pl.BlockSpec((4, _SL, _LN), lambda b: (0, 0, 0))
    spec_bx = pl.BlockSpec((1, _O, 4), lambda b: (b, 0, 0),
                           memory_space=pltpu.SMEM)
    spec_lb = pl.BlockSpec((1, 1, _O), lambda b: (b, 0, 0),
                           memory_space=pltpu.SMEM)
    out_pp = [
        jax.ShapeDtypeStruct((_B, _SL, _LN), jnp.float32),
        jax.ShapeDtypeStruct((_B, _SL, _LN), jnp.int32),
        jax.ShapeDtypeStruct((_B, 1, 128), jnp.float32),
    ]

    def in_spec(k):
        return pl.BlockSpec((1, k, _SL, _LN), lambda b: (b, 0, 0, 0))

    conf_a, bits_a, part_a = pl.pallas_call(
        _arm_body,
        grid=(_B,),
        in_specs=[in_spec(4), in_spec(2), spec_pr, spec_bx, spec_lb],
        out_specs=[spec_pp, spec_pp, spec_part],
        out_shape=out_pp,
    )(al, asr, pr, boxes, labels)

    conf_o, bits_o, part_o = pl.pallas_call(
        _odm_body,
        grid=(_B,),
        in_specs=[in_spec(4), in_spec(2), in_spec(4), in_spec(_C),
                  spec_pr, spec_bx, spec_lb],
        out_specs=[spec_pp, spec_pp, spec_part],
        out_shape=out_pp,
    )(al, asr, ol, osr, pr, boxes, labels)

    negs = _sc_mine(conf_a, bits_a, part_a, conf_o, bits_o, part_o)
    negs_a = negs[:_B * 16]
    negs_o = negs[_B * 16:]

    out = pl.pallas_call(
        _combine_body,
        in_specs=[
            pl.BlockSpec((_B, 1, 128), lambda: (0, 0, 0)),
            pl.BlockSpec((_B, 1, 128), lambda: (0, 0, 0)),
            pl.BlockSpec((_B, 16), lambda: (0, 0)),
            pl.BlockSpec((_B, 16), lambda: (0, 0)),
        ],
        out_specs=pl.BlockSpec((1, 1), lambda: (0, 0), memory_space=pltpu.SMEM),
        out_shape=jax.ShapeDtypeStruct((1, 1), jnp.float32),
    )(part_a, part_o, negs_a.reshape(_B, 16), negs_o.reshape(_B, 16))
    return out.reshape(())


# speedup vs baseline: 1.0209x; 1.0209x over previous
import functools

import jax
import jax.numpy as jnp
from jax import lax
from jax.experimental import pallas as pl
from jax.experimental.pallas import tpu as pltpu
from jax.experimental.pallas import tpu_sc as plsc

_B, _P, _O, _C = 16, 8732, 12, 21
_THR, _RATIO, _THETA, _ALPHA = 0.5, 3, 0.01, 1.0
_SL, _LN = 8, 1152
_PP = _SL * _LN


def _diou(px0, py0, px1, py1, tx0, ty0, tx1, ty1):
    eps = 1e-7
    ix0 = jnp.maximum(px0, tx0)
    iy0 = jnp.maximum(py0, ty0)
    ix1 = jnp.minimum(px1, tx1)
    iy1 = jnp.minimum(py1, ty1)
    inter = jnp.clip(ix1 - ix0, 0.0, None) * jnp.clip(iy1 - iy0, 0.0, None)
    ap = jnp.clip(px1 - px0, 0.0, None) * jnp.clip(py1 - py0, 0.0, None)
    at = jnp.clip(tx1 - tx0, 0.0, None) * jnp.clip(ty1 - ty0, 0.0, None)
    union = ap + at - inter
    iou = inter / (union + eps)
    cpx = (px0 + px1) / 2
    cpy = (py0 + py1) / 2
    ctx = (tx0 + tx1) / 2
    cty = (ty0 + ty1) / 2
    rho2 = (cpx - ctx) ** 2 + (cpy - cty) ** 2
    ex0 = jnp.minimum(px0, tx0)
    ey0 = jnp.minimum(py0, ty0)
    ex1 = jnp.maximum(px1, tx1)
    ey1 = jnp.maximum(py1, ty1)
    c2 = (ex1 - ex0) ** 2 + (ey1 - ey0) ** 2 + eps
    return 1.0 - (iou - rho2 / c2)


def _iotas():
    shp = (_SL, _LN)
    row = lax.broadcasted_iota(jnp.int32, shp, 0)
    coli = lax.broadcasted_iota(jnp.int32, shp, 1)
    pidx = row * _LN + coli
    return pidx, pidx >= _P


def _priors_xy(pr_ref):
    pcx, pcy, pw, ph = pr_ref[0], pr_ref[1], pr_ref[2], pr_ref[3]
    px0 = pcx - pw / 2
    py0 = pcy - ph / 2
    px1 = pcx + pw / 2
    py1 = pcy + ph / 2
    return pcx, pcy, pw, ph, px0, py0, px1, py1


def _arm_decode(al_ref, pr):
    pcx, pcy, pw, ph = pr[0], pr[1], pr[2], pr[3]
    g0, g1, g2, g3 = al_ref[0, 0], al_ref[0, 1], al_ref[0, 2], al_ref[0, 3]
    acx = g0 * pw / 10 + pcx
    acy = g1 * ph / 10 + pcy
    aw = jnp.exp(g2 / 5) * pw
    ah = jnp.exp(g3 / 5) * ph
    return acx - aw / 2, acy - ah / 2, acx + aw / 2, acy + ah / 2


def _read_objs(bx_ref, lb_ref):
    boxes = [[bx_ref[0, o, j] for j in range(4)] for o in range(_O)]
    labels = [lb_ref[0, 0, o] for o in range(_O)]
    return boxes, labels


_NV = _LN // 128


def _bcast_reduce(x, op):
    r = x[:, 0:128]
    for k in range(1, _NV):
        r = op(r, x[:, 128 * k:128 * (k + 1)])
    for s in (1, 2, 4, 8, 16, 32, 64):
        r = op(r, pltpu.roll(r, s, axis=1))
    for s in (1, 2, 4):
        r = op(r, pltpu.roll(r, s, axis=0))
    return jnp.concatenate([r] * _NV, axis=1)


def _run_match(boxes, labels, pidx, x0, y0, x1, y1, thr):
    shp = (_SL, _LN)
    area2 = (x1 - x0) * (y1 - y0)
    best = jnp.full(shp, -1.0, jnp.float32)
    obj = jnp.zeros(shp, jnp.int32)
    mx_l, pf_l = [], []
    for o in range(_O):
        bx0, by0, bx1, by1 = boxes[o]
        a1 = (bx1 - bx0) * (by1 - by0)
        inter = (jnp.clip(jnp.minimum(bx1, x1) - jnp.maximum(bx0, x0), 0.0, None)
                 * jnp.clip(jnp.minimum(by1, y1) - jnp.maximum(by0, y0), 0.0, None))
        ov = inter / jnp.maximum(a1 + area2 - inter, 1e-10)
        upd = ov > best
        best = jnp.where(upd, ov, best)
        obj = jnp.where(upd, o, obj)
        mx = _bcast_reduce(ov, jnp.maximum)
        pf = _bcast_reduce(jnp.where(ov == mx, pidx, _PP), jnp.minimum)
        mx_l.append(mx)
        pf_l.append(pf)
    obj0 = obj
    ofp = best
    for o in range(_O):
        m = mx_l[o] > 0.0
        hit = (pidx == pf_l[o]) & m
        ofp = jnp.where(hit, jnp.maximum(ofp, 1.0), ofp)
        obj = jnp.where(hit, o, jnp.where((pidx == pf_l[o]) & (~m), obj0, obj))
    lfp = jnp.zeros(shp, jnp.int32)
    tx0 = jnp.zeros(shp, jnp.float32)
    ty0 = jnp.zeros(shp, jnp.float32)
    tx1 = jnp.zeros(shp, jnp.float32)
    ty1 = jnp.zeros(shp, jnp.float32)
    for o in range(_O):
        sel = obj == o
        lfp = jnp.where(sel, labels[o], lfp)
        tx0 = jnp.where(sel, boxes[o][0], tx0)
        ty0 = jnp.where(sel, boxes[o][1], ty0)
        tx1 = jnp.where(sel, boxes[o][2], tx1)
        ty1 = jnp.where(sel, boxes[o][3], ty1)
    lfp = jnp.where(ofp < thr, 0, lfp)
    return lfp, (tx0, ty0, tx1, ty1)


def _write_part(part_ref, vals):
    li = lax.broadcasted_iota(jnp.int32, (1, 128), 1)
    acc = jnp.zeros((1, 128), jnp.float32)
    for j, v in enumerate(vals):
        acc = jnp.where(li == j, v, acc)
    part_ref[0] = acc


def _arm_body(al_ref, as_ref, pr_ref, bx_ref, lb_ref,
              conf_ref, bits_ref, part_ref):
    pidx, pad = _iotas()
    pr = _priors_xy(pr_ref)
    px0, py0, px1, py1 = pr[4], pr[5], pr[6], pr[7]
    ax0, ay0, ax1, ay1 = _arm_decode(al_ref, pr)
    boxes, labels = _read_objs(bx_ref, lb_ref)

    lfp_a, tla = _run_match(boxes, labels, pidx, px0, py0, px1, py1,
                            _THR - 0.2)
    pos_a = lfp_a > 0
    n_pos_a = jnp.sum(jnp.where(pos_a, 1.0, 0.0))
    d_a = _diou(ax0, ay0, ax1, ay1, *tla)
    dsum_a = jnp.sum(jnp.where(pos_a, d_a, 0.0))
    s0, s1 = as_ref[0, 0], as_ref[0, 1]
    mx2 = jnp.maximum(s0, s1)
    lse2 = mx2 + jnp.log(jnp.exp(s0 - mx2) + jnp.exp(s1 - mx2))
    ce_a = lse2 - jnp.where(pos_a, s1, s0)
    pos_ce_a = jnp.sum(jnp.where(pos_a, ce_a, 0.0))
    conf_na = jnp.where(pos_a | pad, 0.0, ce_a)
    conf_ref[0] = conf_na
    bits_ref[0] = lax.bitcast_convert_type(conf_na, jnp.int32)
    _write_part(part_ref, [n_pos_a, pos_ce_a, dsum_a])


def _odm_body(al_ref, as_ref, ol_ref, os_ref, pr_ref, bx_ref, lb_ref,
              conf_ref, bits_ref, part_ref):
    pidx, pad = _iotas()
    pr = _priors_xy(pr_ref)
    ax0, ay0, ax1, ay1 = _arm_decode(al_ref, pr)
    boxes, labels = _read_objs(bx_ref, lb_ref)

    lfp_o, tlo = _run_match(boxes, labels, pidx, ax0, ay0, ax1, ay1, _THR)
    s0, s1 = as_ref[0, 0], as_ref[0, 1]
    mx2 = jnp.maximum(s0, s1)
    lse2 = mx2 + jnp.log(jnp.exp(s0 - mx2) + jnp.exp(s1 - mx2))
    easy = jnp.exp(s1 - lse2) < _THETA
    pos_o = (lfp_o > 0) & (~easy)
    n_pos_o = jnp.sum(jnp.where(pos_o, 1.0, 0.0))

    acx2 = (ax0 + ax1) / 2
    acy2 = (ay0 + ay1) / 2
    aw2 = ax1 - ax0
    ah2 = ay1 - ay0
    h0, h1, h2, h3 = ol_ref[0, 0], ol_ref[0, 1], ol_ref[0, 2], ol_ref[0, 3]
    ocx = h0 * aw2 / 10 + acx2
    ocy = h1 * ah2 / 10 + acy2
    ow = jnp.exp(h2 / 5) * aw2
    oh = jnp.exp(h3 / 5) * ah2
    d_o = _diou(ocx - ow / 2, ocy - oh / 2, ocx + ow / 2, ocy + oh / 2, *tlo)
    dsum_o = jnp.sum(jnp.where(pos_o, d_o, 0.0))

    logits = [os_ref[0, c] for c in range(_C)]
    mxc = logits[0]
    for c in range(1, _C):
        mxc = jnp.maximum(mxc, logits[c])
    sume = jnp.exp(logits[0] - mxc)
    for c in range(1, _C):
        sume = sume + jnp.exp(logits[c] - mxc)
    lsec = mxc + jnp.log(sume)
    chosen = jnp.zeros((_SL, _LN), jnp.float32)
    for c in range(_C):
        chosen = jnp.where(lfp_o == c, logits[c], chosen)
    ce_o = lsec - chosen
    pos_ce_o = jnp.sum(jnp.where(pos_o, ce_o, 0.0))
    conf_no = jnp.where(pos_o | easy | pad, 0.0, ce_o)
    conf_ref[0] = conf_no
    bits_ref[0] = lax.bitcast_convert_type(conf_no, jnp.int32)
    _write_part(part_ref, [n_pos_o, pos_ce_o, dsum_o])


_UN = 8


def _sc_mine_body(cf_hbm, ci_hbm, part_hbm, out_hbm, row_f, row_i, part_v, out_v):
    wid = lax.axis_index("s") * 2 + lax.axis_index("c")

    @pl.when(wid < _B)
    def _():
        pltpu.sync_copy(cf_hbm.at[pl.ds(wid * _PP, _PP)], row_f)
        pltpu.sync_copy(ci_hbm.at[pl.ds(wid * _PP, _PP)], row_i)
        pltpu.sync_copy(part_hbm.at[pl.ds(wid * 128, 16)], part_v)
        lane = lax.iota(jnp.int32, 16)

        def _shuf(x, s):
            return x.at[lane ^ s].get(mode="promise_in_bounds")

        def allsum(x):
            for s in (8, 4, 2, 1):
                x = x + _shuf(x, s)
            return x

        def allmax(x):
            for s in (8, 4, 2, 1):
                x = jnp.maximum(x, _shuf(x, s))
            return x

        def tree(xs, op):
            xs = list(xs)
            while len(xs) > 1:
                xs = [op(xs[i], xs[i + 1]) for i in range(0, len(xs) - 1, 2)] + \
                    (xs[-1:] if len(xs) % 2 else [])
            return xs[0]

        kvec = 3.0 * allsum(jnp.where(lane == 0, part_v[...], 0.0))

        zi = jnp.zeros((16,), jnp.int32)
        zf = jnp.zeros((16,), jnp.float32)

        @plsc.parallel_loop(0, _PP, 16 * _UN, carry=(zi,) * _UN)
        def mxs(j, c):
            return tuple(jnp.maximum(c[i], row_i[pl.ds(j + 16 * i, 16)])
                         for i in range(_UN))

        hi = allmax(tree(mxs, jnp.maximum)) + 1
        lo = zi

        def bisect(_, c):
            lo, hi = c
            mid = lo + ((hi - lo) >> 1)

            @plsc.parallel_loop(0, _PP, 16 * _UN, carry=(zf,) * _UN)
            def accs(j, c):
                return tuple(
                    c[i] + jnp.where(row_i[pl.ds(j + 16 * i, 16)] > mid,
                                     1.0, 0.0)
                    for i in range(_UN))

            p = allsum(tree(accs, jnp.add)) < kvec
            return jnp.where(p, lo, mid + 1), jnp.where(p, mid, hi)

        lo, hi = lax.fori_loop(0, 31, bisect, (lo, hi))

        @plsc.parallel_loop(0, _PP, 16 * 4,
                            carry=((zf,) * 4, (zf,) * 4, (zf,) * 4))
        def fin(j, c):
            acc, sab, tac = c
            na, ns, nt = [], [], []
            for i in range(4):
                b = row_i[pl.ds(j + 16 * i, 16)]
                v = row_f[pl.ds(j + 16 * i, 16)]
                gt = b > lo
                na.append(acc[i] + jnp.where(gt, 1.0, 0.0))
                ns.append(sab[i] + jnp.where(gt, v, 0.0))
                nt.append(jnp.maximum(tac[i], jnp.where(b == lo, v, 0.0)))
            return tuple(na), tuple(ns), tuple(nt)

        acc, sab, tac = fin
        t = allmax(tree(tac, jnp.maximum))
        neg = allsum(tree(sab, jnp.add)) + \
            (kvec - allsum(tree(acc, jnp.add))) * t
        out_v[...] = jnp.where(lane == 0, neg, 0.0)
        pltpu.sync_copy(out_v, out_hbm.at[pl.ds(wid * 16, 16)])


def _combine_body(pa_ref, po_ref, na_ref, no_ref, out_ref):
    li = lax.broadcasted_iota(jnp.int32, (_B, 128), 1)

    def col(ref, j):
        return jnp.sum(jnp.where(li == j, ref[:, 0, :], 0.0))

    npa, cepa, da = col(pa_ref, 0), col(pa_ref, 1), col(pa_ref, 2)
    npo, cepo, do_ = col(po_ref, 0), col(po_ref, 1), col(po_ref, 2)
    ci = lax.broadcasted_iota(jnp.int32, (_B, 16), 1)
    neg_a = jnp.sum(jnp.where(ci == 0, na_ref[...], 0.0))
    neg_o = jnp.sum(jnp.where(ci == 0, no_ref[...], 0.0))
    conf_a = (neg_a + cepa) / npa
    loc_a = da / jnp.maximum(npa, 1.0)
    conf_o = (neg_o + cepo) / npo
    loc_o = do_ / jnp.maximum(npo, 1.0)
    out_ref[0, 0] = conf_a + _ALPHA * loc_a + conf_o + _ALPHA * loc_o


def _prep(x):
    b, p, k = x.shape
    xt = jnp.swapaxes(x, 1, 2)
    xt = jnp.pad(xt, ((0, 0), (0, 0), (0, _PP - p)))
    return xt.reshape(b, k, _SL, _LN)


_SC_MESH = dict(core_axis_name="c", subcore_axis_name="s",
                num_cores=2, num_subcores=16)


def _sc_mine(conf, bits, part):
    mesh = plsc.VectorSubcoreMesh(**_SC_MESH)
    return pl.kernel(
        _sc_mine_body,
        out_type=jax.ShapeDtypeStruct((_B * 16,), jnp.float32),
        mesh=mesh,
        scratch_types=[
            pltpu.VMEM((_PP,), jnp.float32),
            pltpu.VMEM((_PP,), jnp.int32),
            pltpu.VMEM((16,), jnp.float32),
            pltpu.VMEM((16,), jnp.float32),
        ],
    )(conf.reshape(-1), bits.reshape(-1), part.reshape(-1))


@jax.jit
def kernel(arm_locs, arm_scores, odm_locs, odm_scores, boxes, labels, priors_cxcy):
    al = _prep(arm_locs)
    asr = _prep(arm_scores)
    ol = _prep(odm_locs)
    osr = _prep(odm_scores)
    pr = jnp.pad(jnp.swapaxes(priors_cxcy, 0, 1),
                 ((0, 0), (0, _PP - _P))).reshape(4, _SL, _LN)
    labels = labels.astype(jnp.int32).reshape(_B, 1, _O)

    spec_pp = pl.BlockSpec((1, _SL, _LN), lambda b: (b, 0, 0))
    spec_part = pl.BlockSpec((1, 1, 128), lambda b: (b, 0, 0))
    spec_pr = pl.BlockSpec((4, _SL, _LN), lambda b: (0, 0, 0))
    spec_bx = pl.BlockSpec((1, _O, 4), lambda b: (b, 0, 0),
                           memory_space=pltpu.SMEM)
    spec_lb = pl.BlockSpec((1, 1, _O), lambda b: (b, 0, 0),
                           memory_space=pltpu.SMEM)
    out_pp = [
        jax.ShapeDtypeStruct((_B, _SL, _LN), jnp.float32),
        jax.ShapeDtypeStruct((_B, _SL, _LN), jnp.int32),
        jax.ShapeDtypeStruct((_B, 1, 128), jnp.float32),
    ]

    def in_spec(k):
        return pl.BlockSpec((1, k, _SL, _LN), lambda b: (b, 0, 0, 0))

    conf_a, bits_a, part_a = pl.pallas_call(
        _arm_body,
        grid=(_B,),
        in_specs=[in_spec(4), in_spec(2), spec_pr, spec_bx, spec_lb],
        out_specs=[spec_pp, spec_pp, spec_part],
        out_shape=out_pp,
    )(al, asr, pr, boxes, labels)

    conf_o, bits_o, part_o = pl.pallas_call(
        _odm_body,
        grid=(_B,),
        in_specs=[in_spec(4), in_spec(2), in_spec(4), in_spec(_C),
                  spec_pr, spec_bx, spec_lb],
        out_specs=[spec_pp, spec_pp, spec_part],
        out_shape=out_pp,
    )(al, asr, ol, osr, pr, boxes, labels)

    negs_a = _sc_mine(conf_a, bits_a, part_a)
    negs_o = _sc_mine(conf_o, bits_o, part_o)

    out = pl.pallas_call(
        _combine_body,
        in_specs=[
            pl.BlockSpec((_B, 1, 128), lambda: (0, 0, 0)),
            pl.BlockSpec((_B, 1, 128), lambda: (0, 0, 0)),
            pl.BlockSpec((_B, 16), lambda: (0, 0)),
            pl.BlockSpec((_B, 16), lambda: (0, 0)),
        ],
        out_specs=pl.BlockSpec((1, 1), lambda: (0, 0), memory_space=pltpu.SMEM),
        out_shape=jax.ShapeDtypeStruct((1, 1), jnp.float32),
    )(part_a, part_o, negs_a.reshape(_B, 16), negs_o.reshape(_B, 16))
    return out.reshape(())
